# baseline (speedup 1.0000x reference)
import functools

import jax
import jax.numpy as jnp
from jax import lax
from jax.experimental import pallas as pl
from jax.experimental.pallas import tpu as pltpu
from jax.experimental.pallas import tpu_sc as plsc

BATCH = 4
SEQ_LEN = 8192
EMBED_DIM = 1024

NUM_CORES = 2
NUM_SUBCORES = 16
NUM_WORKERS = NUM_CORES * NUM_SUBCORES

TOTAL = BATCH * SEQ_LEN * EMBED_DIM
TABLE_TOTAL = SEQ_LEN * EMBED_DIM
PER_WORKER = TOTAL // NUM_WORKERS
WORKERS_PER_BATCH = TABLE_TOTAL // PER_WORKER

PIECE = 32768
NPIECES = PER_WORKER // PIECE
LANES = 16
UNROLL = 16


@functools.partial(
    pl.kernel,
    out_type=jax.ShapeDtypeStruct((TOTAL,), jnp.float32),
    scratch_types=[
        pltpu.VMEM((PIECE,), jnp.float32),
        pltpu.VMEM((PIECE,), jnp.float32),
    ],
    mesh=plsc.VectorSubcoreMesh(core_axis_name="c", subcore_axis_name="s"),
)
def _sc_add(x_hbm, t_hbm, o_hbm, vx, vt):
    w = lax.axis_index("s") * NUM_CORES + lax.axis_index("c")
    x_base = w * PER_WORKER
    t_base = (w % WORKERS_PER_BATCH) * PER_WORKER

    def piece_body(p, carry):
        xo = x_base + p * PIECE
        to = t_base + p * PIECE
        pltpu.sync_copy(x_hbm.at[pl.ds(xo, PIECE)], vx)
        pltpu.sync_copy(t_hbm.at[pl.ds(to, PIECE)], vt)

        def add_body(i, c):
            base = i * (LANES * UNROLL)
            for k in range(UNROLL):
                off = base + k * LANES
                vx[pl.ds(off, LANES)] = vx[pl.ds(off, LANES)] + vt[pl.ds(off, LANES)]
            return c

        lax.fori_loop(0, PIECE // (LANES * UNROLL), add_body, 0)
        pltpu.sync_copy(vx, o_hbm.at[pl.ds(xo, PIECE)])
        return carry

    lax.fori_loop(0, NPIECES, piece_body, 0)


def kernel(x, pos_table):
    out_flat = _sc_add(x.reshape(-1), pos_table.reshape(-1))
    return out_flat.reshape(x.shape)


# speedup vs baseline: 1.1202x; 1.1202x over previous
import functools

import jax
import jax.numpy as jnp
from jax import lax
from jax.experimental import pallas as pl
from jax.experimental.pallas import tpu as pltpu
from jax.experimental.pallas import tpu_sc as plsc

BATCH = 4
SEQ_LEN = 8192
EMBED_DIM = 1024

NUM_CORES = 2
NUM_SUBCORES = 16
NUM_WORKERS = NUM_CORES * NUM_SUBCORES

TABLE_TOTAL = SEQ_LEN * EMBED_DIM
TOTAL = BATCH * TABLE_TOTAL
TW = TABLE_TOTAL // NUM_WORKERS

PIECE = 4096
NPIECES = TW // PIECE
LANES = 16
UNROLL = 8
CHUNK_ITERS = PIECE // (LANES * UNROLL)


@functools.partial(
    pl.kernel,
    out_type=jax.ShapeDtypeStruct((TOTAL,), jnp.float32),
    scratch_types=[
        pltpu.VMEM((PIECE,), jnp.float32),
        pltpu.VMEM((PIECE,), jnp.float32),
        pltpu.VMEM((BATCH, PIECE), jnp.float32),
        pltpu.VMEM((BATCH, PIECE), jnp.float32),
        pltpu.VMEM((BATCH, PIECE), jnp.float32),
        pltpu.VMEM((BATCH, PIECE), jnp.float32),
        pltpu.SemaphoreType.DMA,
        pltpu.SemaphoreType.DMA,
        pltpu.SemaphoreType.DMA,
        pltpu.SemaphoreType.DMA,
    ],
    mesh=plsc.VectorSubcoreMesh(core_axis_name="c", subcore_axis_name="s"),
)
def _sc_add(x_hbm, t_hbm, o_hbm, vt0, vt1, vx0, vx1, vy0, vy1,
            in_sem0, in_sem1, out_sem0, out_sem1):
    w = lax.axis_index("s") * NUM_CORES + lax.axis_index("c")
    t_base = w * TW

    sets = ((vt0, vx0, vy0, in_sem0, out_sem0),
            (vt1, vx1, vy1, in_sem1, out_sem1))

    def start_in(p, vt, vx, in_sem):
        to = t_base + p * PIECE
        pltpu.async_copy(t_hbm.at[pl.ds(to, PIECE)], vt, in_sem)
        for b in range(BATCH):
            xo = b * TABLE_TOTAL + to
            pltpu.async_copy(x_hbm.at[pl.ds(xo, PIECE)], vx.at[b], in_sem)

    def wait_in(p, vt, vx, in_sem):
        to = t_base + p * PIECE
        pltpu.make_async_copy(t_hbm.at[pl.ds(to, PIECE)], vt, in_sem).wait()
        for b in range(BATCH):
            xo = b * TABLE_TOTAL + to
            pltpu.make_async_copy(x_hbm.at[pl.ds(xo, PIECE)], vx.at[b], in_sem).wait()

    def start_out(p, vy, out_sem):
        to = t_base + p * PIECE
        for b in range(BATCH):
            xo = b * TABLE_TOTAL + to
            pltpu.async_copy(vy.at[b], o_hbm.at[pl.ds(xo, PIECE)], out_sem)

    def wait_out(p, vy, out_sem):
        to = t_base + p * PIECE
        for b in range(BATCH):
            xo = b * TABLE_TOTAL + to
            pltpu.make_async_copy(vy.at[b], o_hbm.at[pl.ds(xo, PIECE)], out_sem).wait()

    def compute(vt, vx, vy):
        def add_body(i, c):
            base = i * (LANES * UNROLL)
            for k in range(UNROLL):
                off = base + k * LANES
                t = vt[pl.ds(off, LANES)]
                for b in range(BATCH):
                    vy[b, pl.ds(off, LANES)] = vx[b, pl.ds(off, LANES)] + t
            return c
        lax.fori_loop(0, CHUNK_ITERS, add_body, 0)

    start_in(0, sets[0][0], sets[0][1], sets[0][3])
    start_in(1, sets[1][0], sets[1][1], sets[1][3])

    def outer(i, carry):
        for j in range(2):
            vt, vx, vy, in_sem, out_sem = sets[j]
            p = 2 * i + j
            wait_in(p, vt, vx, in_sem)

            @pl.when(i > 0)
            def _():
                wait_out(p - 2, vy, out_sem)

            compute(vt, vx, vy)
            start_out(p, vy, out_sem)

            @pl.when(p + 2 < NPIECES)
            def _():
                start_in(p + 2, vt, vx, in_sem)
        return carry

    lax.fori_loop(0, NPIECES // 2, outer, 0)

    wait_out(NPIECES - 2, sets[0][2], sets[0][4])
    wait_out(NPIECES - 1, sets[1][2], sets[1][4])


def kernel(x, pos_table):
    out_flat = _sc_add(x.reshape(-1), pos_table.reshape(-1))
    return out_flat.reshape(x.shape)


# speedup vs baseline: 1.2472x; 1.1133x over previous
import functools

import jax
import jax.numpy as jnp
from jax import lax
from jax.experimental import pallas as pl
from jax.experimental.pallas import tpu as pltpu
from jax.experimental.pallas import tpu_sc as plsc

BATCH = 4
SEQ_LEN = 8192
EMBED_DIM = 1024

NUM_CORES = 2
NUM_SUBCORES = 16
NUM_WORKERS = NUM_CORES * NUM_SUBCORES

TABLE_TOTAL = SEQ_LEN * EMBED_DIM
TOTAL = BATCH * TABLE_TOTAL
PER_WORKER = TOTAL // NUM_WORKERS
WORKERS_PER_BATCH = TABLE_TOTAL // PER_WORKER

PIECE = 16384
NPIECES = PER_WORKER // PIECE
NSETS = 3
LANES = 16
UNROLL = 8
CHUNK_ITERS = PIECE // (LANES * UNROLL)


@functools.partial(
    pl.kernel,
    out_type=jax.ShapeDtypeStruct((TOTAL,), jnp.float32),
    scratch_types=[
        pltpu.VMEM((PIECE,), jnp.float32),
        pltpu.VMEM((PIECE,), jnp.float32),
        pltpu.VMEM((PIECE,), jnp.float32),
        pltpu.VMEM((PIECE,), jnp.float32),
        pltpu.VMEM((PIECE,), jnp.float32),
        pltpu.VMEM((PIECE,), jnp.float32),
        pltpu.SemaphoreType.DMA,
        pltpu.SemaphoreType.DMA,
        pltpu.SemaphoreType.DMA,
        pltpu.SemaphoreType.DMA,
        pltpu.SemaphoreType.DMA,
        pltpu.SemaphoreType.DMA,
    ],
    mesh=plsc.VectorSubcoreMesh(core_axis_name="c", subcore_axis_name="s"),
)
def _sc_add(x_hbm, t_hbm, o_hbm, vx0, vx1, vx2, vt0, vt1, vt2,
            isem0, isem1, isem2, osem0, osem1, osem2):
    w = lax.axis_index("s") * NUM_CORES + lax.axis_index("c")
    x_base = w * PER_WORKER
    t_base = (w % WORKERS_PER_BATCH) * PER_WORKER
    vxs = (vx0, vx1, vx2)
    vts = (vt0, vt1, vt2)
    isems = (isem0, isem1, isem2)
    osems = (osem0, osem1, osem2)

    def start_in(p, j):
        off = p * PIECE
        pltpu.async_copy(x_hbm.at[pl.ds(x_base + off, PIECE)], vxs[j], isems[j])
        pltpu.async_copy(t_hbm.at[pl.ds(t_base + off, PIECE)], vts[j], isems[j])

    def wait_in(p, j):
        off = p * PIECE
        pltpu.make_async_copy(x_hbm.at[pl.ds(x_base + off, PIECE)], vxs[j], isems[j]).wait()
        pltpu.make_async_copy(t_hbm.at[pl.ds(t_base + off, PIECE)], vts[j], isems[j]).wait()

    def start_out(p, j):
        off = p * PIECE
        pltpu.async_copy(vxs[j], o_hbm.at[pl.ds(x_base + off, PIECE)], osems[j])

    def wait_out(p, j):
        off = p * PIECE
        pltpu.make_async_copy(vxs[j], o_hbm.at[pl.ds(x_base + off, PIECE)], osems[j]).wait()

    def compute(j):
        def add_body(i, c):
            base = i * (LANES * UNROLL)
            for k in range(UNROLL):
                off = base + k * LANES
                vxs[j][pl.ds(off, LANES)] = vxs[j][pl.ds(off, LANES)] + vts[j][pl.ds(off, LANES)]
            return c
        lax.fori_loop(0, CHUNK_ITERS, add_body, 0)

    def process(p, j):
        jn = (j + 1) % NSETS
        if isinstance(p, int):
            if p >= 2:
                wait_out(p - 2, jn)
            if p + 1 < NPIECES:
                start_in(p + 1, jn)
        else:
            @pl.when(p >= 2)
            def _():
                wait_out(p - 2, jn)

            @pl.when(p + 1 < NPIECES)
            def _():
                start_in(p + 1, jn)

        wait_in(p, j)
        compute(j)
        start_out(p, j)

    start_in(0, 0)

    def outer(i, carry):
        p0 = 3 * i
        for j in range(NSETS):
            process(p0 + j, j)
        return carry

    lax.fori_loop(0, (NPIECES - 1) // NSETS, outer, 0)
    process(NPIECES - 1, 0)

    wait_out(NPIECES - 2, 2)
    wait_out(NPIECES - 1, 0)


def kernel(x, pos_table):
    out_flat = _sc_add(x.reshape(-1), pos_table.reshape(-1))
    return out_flat.reshape(x.shape)
